# baseline (speedup 1.0000x reference)
import jax
import jax.numpy as jnp
from jax.experimental import pallas as pl
from jax.experimental.pallas import tpu as pltpu

B, S, D = 4, 2048, 1024
H = S // 2
NSTEP = 2 * B


def _body(x_hbm, emb_hbm, o_hbm, eb, *bufs_and_sems):
    xbufs = bufs_and_sems[:NSTEP]
    se = bufs_and_sems[NSTEP]
    si = bufs_and_sems[NSTEP + 1:2 * NSTEP + 1]
    so = bufs_and_sems[2 * NSTEP + 1:]

    def xcopy(k):
        b, h = k // 2, k % 2
        return pltpu.make_async_copy(
            x_hbm.at[b, pl.ds(h * H, H)], xbufs[k], si[k])

    def ocopy(k):
        b, h = k // 2, k % 2
        return pltpu.make_async_copy(
            xbufs[k], o_hbm.at[b, pl.ds(h * H, H)], so[k])

    ecopy = pltpu.make_async_copy(emb_hbm, eb, se)
    ecopy.start()
    for k in range(NSTEP):
        xcopy(k).start()
    ecopy.wait()
    for k in range(NSTEP):
        xcopy(k).wait()
        xb = xbufs[k]
        xb[...] = xb[...] + eb[pl.ds((k % 2) * H, H), :]
        ocopy(k).start()
    for k in range(NSTEP):
        ocopy(k).wait()


def kernel(x, emb_weight):
    return pl.pallas_call(
        _body,
        in_specs=[
            pl.BlockSpec(memory_space=pl.ANY),
            pl.BlockSpec(memory_space=pl.ANY),
        ],
        out_specs=pl.BlockSpec(memory_space=pl.ANY),
        out_shape=jax.ShapeDtypeStruct(x.shape, x.dtype),
        scratch_shapes=(
            [pltpu.VMEM((S, D), jnp.float32)]
            + [pltpu.VMEM((H, D), jnp.float32) for _ in range(NSTEP)]
            + [pltpu.SemaphoreType.DMA for _ in range(2 * NSTEP + 1)]
        ),
    )(x, emb_weight)


# speedup vs baseline: 1.0212x; 1.0212x over previous
import jax
import jax.numpy as jnp
from jax.experimental import pallas as pl
from jax.experimental.pallas import tpu as pltpu

B, S, D = 4, 2048, 1024
H = S // 2


def _body(x_hbm, emb_hbm, o_hbm, eb, xb0, xb1, xb2, xb3,
          se, si0, si1, si2, si3, *so):
    xbufs = (xb0, xb1, xb2, xb3)
    si = (si0, si1, si2, si3)

    def xcopy(b):
        return pltpu.make_async_copy(x_hbm.at[b], xbufs[b], si[b])

    def ocopy(b, h):
        return pltpu.make_async_copy(
            xbufs[b].at[pl.ds(h * H, H)],
            o_hbm.at[b, pl.ds(h * H, H)],
            so[2 * b + h],
        )

    ecopy = pltpu.make_async_copy(emb_hbm, eb, se)
    ecopy.start()
    for b in range(B):
        xcopy(b).start()
    ecopy.wait()
    for b in range(B):
        xcopy(b).wait()
        xb = xbufs[b]
        for h in range(2):
            rs = pl.ds(h * H, H)
            xb[rs, :] = xb[rs, :] + eb[rs, :]
            ocopy(b, h).start()
    for b in range(B):
        for h in range(2):
            ocopy(b, h).wait()


def kernel(x, emb_weight):
    return pl.pallas_call(
        _body,
        in_specs=[
            pl.BlockSpec(memory_space=pl.ANY),
            pl.BlockSpec(memory_space=pl.ANY),
        ],
        out_specs=pl.BlockSpec(memory_space=pl.ANY),
        out_shape=jax.ShapeDtypeStruct(x.shape, x.dtype),
        scratch_shapes=(
            [pltpu.VMEM((S, D), jnp.float32) for _ in range(5)]
            + [pltpu.SemaphoreType.DMA for _ in range(13)]
        ),
    )(x, emb_weight)
